# baseline (speedup 1.0000x reference)
import functools

import jax
import jax.numpy as jnp
from jax.experimental import pallas as pl
from jax.experimental.pallas import tpu as pltpu


def _merge_sorted(a, b):
    k2 = len(a)
    out = []
    for j in range(1, k2 + 1):
        terms = []
        for t in range(j + 1):
            if t == 0:
                terms.append(b[j - 1])
            elif t == j:
                terms.append(a[j - 1])
            else:
                terms.append(jnp.maximum(a[t - 1], b[j - t - 1]))
        r = terms[0]
        for tm in terms[1:]:
            r = jnp.minimum(r, tm)
        out.append(r)
    return out


def _normalize_kernel(h_ref, hn_ref):
    h = h_ref[...]
    norm = jnp.sqrt(jnp.sum(h * h, axis=1, keepdims=True))
    hn_ref[...] = h / (norm + 1e-8)


def _router_kernel(h_ref, hn_full_ref, hn_blk_ref, wc_ref, bc_ref, w1_ref,
                   b1_ref, w2_ref, b2_ref, alpha_ref, w_out_ref, l_out_ref,
                   dscr_ref, cscr_ref, *, block_rows, k):
    n = hn_full_ref.shape[0]
    k2 = k + 1
    h_blk = h_ref[...]
    hn_blk = hn_blk_ref[...]

    dscr_ref[...] = jax.lax.dot_general(
        hn_blk, hn_full_ref[...],
        dimension_numbers=(((1,), (1,)), ((), ())),
        preferred_element_type=jnp.float32)

    n_chunks = n // 128
    unroll = 64
    while n_chunks % unroll:
        unroll //= 2
    for rg in range(block_rows // 8):
        r0 = rg * 8

        def body(ci, regs, r0=r0):
            regs = list(regs)
            for u in range(unroll):
                x = dscr_ref[pl.ds(r0, 8),
                             pl.ds((ci * unroll + u) * 128, 128)]
                new = [jnp.maximum(regs[0], x)]
                for j in range(1, k2):
                    new.append(jnp.maximum(regs[j],
                                           jnp.minimum(regs[j - 1], x)))
                regs = new
            return tuple(regs)

        regs = jax.lax.fori_loop(
            0, n_chunks // unroll, body,
            tuple(jnp.full((8, 128), -jnp.inf, jnp.float32)
                  for _ in range(k2)))
        for j in range(k2):
            cscr_ref[pl.ds(r0, 8), pl.ds(j * 128, 128)] = regs[j]

    dist = 1.0 - cscr_ref[...]

    kp1 = float(k + 1)
    taken = jnp.zeros((block_rows, 1), jnp.float32)
    ms, ovs = [], []
    max_last = jnp.zeros((block_rows, 1), jnp.float32)
    for _ in range(k + 1):
        active = taken < kp1
        m = jnp.min(dist, axis=1, keepdims=True)
        m = jnp.where(active, m, 0.0)
        eq = dist == m
        cnt = jnp.sum(eq.astype(jnp.float32), axis=1, keepdims=True)
        dist = jnp.where(eq & active, jnp.inf, dist)
        lo = jnp.maximum(taken + 1.0, 2.0)
        hi = jnp.minimum(taken + cnt, kp1)
        ov = jnp.where(active, jnp.clip(hi - lo + 1.0, 0.0, float(k)), 0.0)
        ms.append(m)
        ovs.append(ov)
        max_last = jnp.where(active & (taken + cnt >= kp1), m, max_last)
        taken = taken + jnp.where(active, cnt, 0.0)

    ksum = jnp.zeros((block_rows, 1), jnp.float32)
    for m, ov in zip(ms, ovs):
        ksum += ov * m
    mean = ksum / float(k)
    var = jnp.zeros((block_rows, 1), jnp.float32)
    for m, ov in zip(ms, ovs):
        dmu = m - mean
        var += ov * dmu * dmu
    std = jnp.sqrt(var / float(k - 1))
    outlier = max_last / (mean + 1e-8)

    w1 = w1_ref[...]
    hid = (std * w1[:, 0][None, :] + mean * w1[:, 1][None, :]
           + outlier * w1[:, 2][None, :] + b1_ref[...])
    hid = jnp.maximum(hid, 0.0)
    topo = jax.lax.dot_general(
        hid, w2_ref[...],
        dimension_numbers=(((1,), (1,)), ((), ())),
        preferred_element_type=jnp.float32) + b2_ref[...]

    content = jax.lax.dot_general(
        h_blk, wc_ref[...],
        dimension_numbers=(((1,), (1,)), ((), ())),
        preferred_element_type=jnp.float32) + bc_ref[...]

    mix = jax.nn.sigmoid(alpha_ref[0, 0])
    logits = mix * content + (1.0 - mix) * topo
    zmax = jnp.max(logits, axis=1, keepdims=True)
    ez = jnp.exp(logits - zmax)
    weights = ez / jnp.sum(ez, axis=1, keepdims=True)

    w_out_ref[...] = weights
    l_out_ref[...] = logits


def kernel(H, Wc, bc, W1, b1, W2, b2, alpha):
    b, s, d = H.shape
    n = b * s
    ng = Wc.shape[0]
    nh = W1.shape[0]
    k = min(8, n - 1)

    block_rows = 1024
    while n % block_rows != 0:
        block_rows //= 2
    grid = n // block_rows

    hf = H.reshape(n, d)

    norm_rows = min(n, 1024)
    hn = pl.pallas_call(
        _normalize_kernel,
        grid=(n // norm_rows,),
        in_specs=[pl.BlockSpec((norm_rows, d), lambda i: (i, 0))],
        out_specs=pl.BlockSpec((norm_rows, d), lambda i: (i, 0)),
        out_shape=jax.ShapeDtypeStruct((n, d), jnp.float32),
    )(hf)

    body = functools.partial(_router_kernel, block_rows=block_rows, k=k)
    weights, logits = pl.pallas_call(
        body,
        grid=(grid,),
        in_specs=[
            pl.BlockSpec((block_rows, d), lambda i: (i, 0)),
            pl.BlockSpec((n, d), lambda i: (0, 0)),
            pl.BlockSpec((block_rows, d), lambda i: (i, 0)),
            pl.BlockSpec((ng, d), lambda i: (0, 0)),
            pl.BlockSpec((1, ng), lambda i: (0, 0)),
            pl.BlockSpec((nh, 3), lambda i: (0, 0)),
            pl.BlockSpec((1, nh), lambda i: (0, 0)),
            pl.BlockSpec((ng, nh), lambda i: (0, 0)),
            pl.BlockSpec((1, ng), lambda i: (0, 0)),
            pl.BlockSpec((1, 1), lambda i: (0, 0)),
        ],
        out_specs=[
            pl.BlockSpec((block_rows, ng), lambda i: (i, 0)),
            pl.BlockSpec((block_rows, ng), lambda i: (i, 0)),
        ],
        out_shape=[
            jax.ShapeDtypeStruct((n, ng), jnp.float32),
            jax.ShapeDtypeStruct((n, ng), jnp.float32),
        ],
        scratch_shapes=[
            pltpu.VMEM((block_rows, n), jnp.float32),
            pltpu.VMEM((block_rows, (k + 1) * 128), jnp.float32),
        ],
    )(hf, hn, hn, Wc, bc.reshape(1, ng), W1, b1.reshape(1, nh), W2,
      b2.reshape(1, ng), alpha.reshape(1, 1))

    return weights.reshape(b, s, ng), logits.reshape(b, s, ng)


# speedup vs baseline: 1.2337x; 1.2337x over previous
import functools

import jax
import jax.numpy as jnp
from jax.experimental import pallas as pl
from jax.experimental.pallas import tpu as pltpu


def _merge_sorted(a, b):
    k2 = len(a)
    out = []
    for j in range(1, k2 + 1):
        terms = []
        for t in range(j + 1):
            if t == 0:
                terms.append(b[j - 1])
            elif t == j:
                terms.append(a[j - 1])
            else:
                terms.append(jnp.maximum(a[t - 1], b[j - t - 1]))
        r = terms[0]
        for tm in terms[1:]:
            r = jnp.minimum(r, tm)
        out.append(r)
    return out


def _normalize_kernel(h_ref, hn_ref):
    h = h_ref[...]
    norm = jnp.sqrt(jnp.sum(h * h, axis=1, keepdims=True))
    hn_ref[...] = h / (norm + 1e-8)


def _router_kernel(h_ref, hn_full_ref, hn_blk_ref, wc_ref, bc_ref, w1_ref,
                   b1_ref, w2_ref, b2_ref, alpha_ref, w_out_ref, l_out_ref,
                   dscr_ref, cscr_ref, *, block_rows, k):
    n = hn_full_ref.shape[0]
    k2 = k + 1
    h_blk = h_ref[...]
    hn_blk = hn_blk_ref[...]

    dscr_ref[...] = jax.lax.dot_general(
        hn_blk, hn_full_ref[...],
        dimension_numbers=(((1,), (1,)), ((), ())),
        preferred_element_type=jnp.float32)

    n_chunks = n // 128
    unroll = 64
    while n_chunks % unroll:
        unroll //= 2
    for rg in range(block_rows // 8):
        r0 = rg * 8

        def body(ci, regs, r0=r0):
            regs = list(regs)
            for u in range(unroll):
                x = dscr_ref[pl.ds(r0, 8),
                             pl.ds((ci * unroll + u) * 128, 128)]
                new = [jnp.maximum(regs[0], x)]
                for j in range(1, k2):
                    new.append(jnp.maximum(regs[j],
                                           jnp.minimum(regs[j - 1], x)))
                regs = new
            return tuple(regs)

        regs = jax.lax.fori_loop(
            0, n_chunks // unroll, body,
            tuple(jnp.full((8, 128), -jnp.inf, jnp.float32)
                  for _ in range(k2)))
        for j in range(k2):
            cscr_ref[pl.ds(r0, 8), pl.ds(j * 128, 128)] = regs[j]

    dist = 1.0 - cscr_ref[...]

    kp1 = float(k + 1)
    taken = jnp.zeros((block_rows, 1), jnp.float32)
    ms, ovs = [], []
    max_last = jnp.zeros((block_rows, 1), jnp.float32)
    for _ in range(k + 1):
        active = taken < kp1
        m = jnp.min(dist, axis=1, keepdims=True)
        m = jnp.where(active, m, 0.0)
        eq = dist == m
        cnt = jnp.sum(eq.astype(jnp.float32), axis=1, keepdims=True)
        dist = jnp.where(eq & active, jnp.inf, dist)
        lo = jnp.maximum(taken + 1.0, 2.0)
        hi = jnp.minimum(taken + cnt, kp1)
        ov = jnp.where(active, jnp.clip(hi - lo + 1.0, 0.0, float(k)), 0.0)
        ms.append(m)
        ovs.append(ov)
        max_last = jnp.where(active & (taken + cnt >= kp1), m, max_last)
        taken = taken + jnp.where(active, cnt, 0.0)

    ksum = jnp.zeros((block_rows, 1), jnp.float32)
    for m, ov in zip(ms, ovs):
        ksum += ov * m
    mean = ksum / float(k)
    var = jnp.zeros((block_rows, 1), jnp.float32)
    for m, ov in zip(ms, ovs):
        dmu = m - mean
        var += ov * dmu * dmu
    std = jnp.sqrt(var / float(k - 1))
    outlier = max_last / (mean + 1e-8)

    w1 = w1_ref[...]
    hid = (std * w1[:, 0][None, :] + mean * w1[:, 1][None, :]
           + outlier * w1[:, 2][None, :] + b1_ref[...])
    hid = jnp.maximum(hid, 0.0)
    topo = jax.lax.dot_general(
        hid, w2_ref[...],
        dimension_numbers=(((1,), (1,)), ((), ())),
        preferred_element_type=jnp.float32) + b2_ref[...]

    content = jax.lax.dot_general(
        h_blk, wc_ref[...],
        dimension_numbers=(((1,), (1,)), ((), ())),
        preferred_element_type=jnp.float32) + bc_ref[...]

    mix = jax.nn.sigmoid(alpha_ref[0, 0])
    logits = mix * content + (1.0 - mix) * topo
    zmax = jnp.max(logits, axis=1, keepdims=True)
    ez = jnp.exp(logits - zmax)
    weights = ez / jnp.sum(ez, axis=1, keepdims=True)

    w_out_ref[...] = weights
    l_out_ref[...] = logits


def kernel(H, Wc, bc, W1, b1, W2, b2, alpha):
    b, s, d = H.shape
    n = b * s
    ng = Wc.shape[0]
    nh = W1.shape[0]
    k = min(8, n - 1)

    block_rows = 512
    while n % block_rows != 0:
        block_rows //= 2
    grid = n // block_rows

    hf = H.reshape(n, d)

    norm_rows = min(n, 1024)
    hn = pl.pallas_call(
        _normalize_kernel,
        grid=(n // norm_rows,),
        in_specs=[pl.BlockSpec((norm_rows, d), lambda i: (i, 0))],
        out_specs=pl.BlockSpec((norm_rows, d), lambda i: (i, 0)),
        out_shape=jax.ShapeDtypeStruct((n, d), jnp.float32),
    )(hf)

    body = functools.partial(_router_kernel, block_rows=block_rows, k=k)
    weights, logits = pl.pallas_call(
        body,
        grid=(grid,),
        in_specs=[
            pl.BlockSpec((block_rows, d), lambda i: (i, 0)),
            pl.BlockSpec((n, d), lambda i: (0, 0)),
            pl.BlockSpec((block_rows, d), lambda i: (i, 0)),
            pl.BlockSpec((ng, d), lambda i: (0, 0)),
            pl.BlockSpec((1, ng), lambda i: (0, 0)),
            pl.BlockSpec((nh, 3), lambda i: (0, 0)),
            pl.BlockSpec((1, nh), lambda i: (0, 0)),
            pl.BlockSpec((ng, nh), lambda i: (0, 0)),
            pl.BlockSpec((1, ng), lambda i: (0, 0)),
            pl.BlockSpec((1, 1), lambda i: (0, 0)),
        ],
        out_specs=[
            pl.BlockSpec((block_rows, ng), lambda i: (i, 0)),
            pl.BlockSpec((block_rows, ng), lambda i: (i, 0)),
        ],
        out_shape=[
            jax.ShapeDtypeStruct((n, ng), jnp.float32),
            jax.ShapeDtypeStruct((n, ng), jnp.float32),
        ],
        scratch_shapes=[
            pltpu.VMEM((block_rows, n), jnp.float32),
            pltpu.VMEM((block_rows, (k + 1) * 128), jnp.float32),
        ],
    )(hf, hn, hn, Wc, bc.reshape(1, ng), W1, b1.reshape(1, nh), W2,
      b2.reshape(1, ng), alpha.reshape(1, 1))

    return weights.reshape(b, s, ng), logits.reshape(b, s, ng)


# speedup vs baseline: 1.3346x; 1.0818x over previous
import functools

import jax
import jax.numpy as jnp
from jax.experimental import pallas as pl
from jax.experimental.pallas import tpu as pltpu


def _merge_sorted(a, b):
    k2 = len(a)
    out = []
    for j in range(1, k2 + 1):
        terms = []
        for t in range(j + 1):
            if t == 0:
                terms.append(b[j - 1])
            elif t == j:
                terms.append(a[j - 1])
            else:
                terms.append(jnp.maximum(a[t - 1], b[j - t - 1]))
        r = terms[0]
        for tm in terms[1:]:
            r = jnp.minimum(r, tm)
        out.append(r)
    return out


def _normalize_kernel(h_ref, hn_ref):
    h = h_ref[...]
    norm = jnp.sqrt(jnp.sum(h * h, axis=1, keepdims=True))
    hn_ref[...] = h / (norm + 1e-8)


def _router_kernel(h_ref, hn_full_ref, hn_blk_ref, wc_ref, bc_ref, w1_ref,
                   b1_ref, w2_ref, b2_ref, alpha_ref, w_out_ref, l_out_ref,
                   dscr_ref, cscr_ref, *, block_rows, k):
    n = hn_full_ref.shape[0]
    k2 = k + 1
    h_blk = h_ref[...]
    hn_blk = hn_blk_ref[...]

    dscr_ref[...] = jax.lax.dot_general(
        hn_blk, hn_full_ref[...],
        dimension_numbers=(((1,), (1,)), ((), ())),
        preferred_element_type=jnp.float32)

    n_chunks = n // 128
    unroll = 64
    while n_chunks % unroll:
        unroll //= 2
    for rg in range(block_rows // 8):
        r0 = rg * 8

        def body(ci, regs, r0=r0):
            regs = list(regs)
            for u0 in range(0, unroll, 8):
                xs = [dscr_ref[pl.ds(r0, 8),
                               pl.ds((ci * unroll + u0 + u) * 128, 128)]
                      for u in range(min(8, unroll - u0))]
                if len(xs) == 8:
                    for (p, q) in ((0, 1), (2, 3), (4, 5), (6, 7),
                                   (0, 2), (1, 3), (4, 6), (5, 7),
                                   (1, 2), (5, 6),
                                   (0, 4), (1, 5), (2, 6), (3, 7),
                                   (2, 4), (3, 5),
                                   (1, 2), (3, 4), (5, 6)):
                        hi = jnp.maximum(xs[p], xs[q])
                        lo = jnp.minimum(xs[p], xs[q])
                        xs[p], xs[q] = hi, lo
                    new = []
                    for j in range(1, k2 + 1):
                        terms = []
                        if j <= 8:
                            terms.append(xs[j - 1])
                        terms.append(regs[j - 1])
                        for t in range(1, min(j - 1, 8) + 1):
                            if t == j:
                                continue
                            terms.append(jnp.minimum(xs[t - 1],
                                                     regs[j - t - 1]))
                        r = terms[0]
                        for tm in terms[1:]:
                            r = jnp.maximum(r, tm)
                        new.append(r)
                    regs = new
                else:
                    for x in xs:
                        new = [jnp.maximum(regs[0], x)]
                        for j in range(1, k2):
                            new.append(jnp.maximum(regs[j],
                                                   jnp.minimum(regs[j - 1],
                                                               x)))
                        regs = new
            return tuple(regs)

        regs = jax.lax.fori_loop(
            0, n_chunks // unroll, body,
            tuple(jnp.full((8, 128), -jnp.inf, jnp.float32)
                  for _ in range(k2)))
        for j in range(k2):
            cscr_ref[pl.ds(r0, 8), pl.ds(j * 128, 128)] = regs[j]

    dist = 1.0 - cscr_ref[...]

    kp1 = float(k + 1)
    taken = jnp.zeros((block_rows, 1), jnp.float32)
    ms, ovs = [], []
    max_last = jnp.zeros((block_rows, 1), jnp.float32)
    for _ in range(k + 1):
        active = taken < kp1
        m = jnp.min(dist, axis=1, keepdims=True)
        m = jnp.where(active, m, 0.0)
        eq = dist == m
        cnt = jnp.sum(eq.astype(jnp.float32), axis=1, keepdims=True)
        dist = jnp.where(eq & active, jnp.inf, dist)
        lo = jnp.maximum(taken + 1.0, 2.0)
        hi = jnp.minimum(taken + cnt, kp1)
        ov = jnp.where(active, jnp.clip(hi - lo + 1.0, 0.0, float(k)), 0.0)
        ms.append(m)
        ovs.append(ov)
        max_last = jnp.where(active & (taken + cnt >= kp1), m, max_last)
        taken = taken + jnp.where(active, cnt, 0.0)

    ksum = jnp.zeros((block_rows, 1), jnp.float32)
    for m, ov in zip(ms, ovs):
        ksum += ov * m
    mean = ksum / float(k)
    var = jnp.zeros((block_rows, 1), jnp.float32)
    for m, ov in zip(ms, ovs):
        dmu = m - mean
        var += ov * dmu * dmu
    std = jnp.sqrt(var / float(k - 1))
    outlier = max_last / (mean + 1e-8)

    w1 = w1_ref[...]
    hid = (std * w1[:, 0][None, :] + mean * w1[:, 1][None, :]
           + outlier * w1[:, 2][None, :] + b1_ref[...])
    hid = jnp.maximum(hid, 0.0)
    topo = jax.lax.dot_general(
        hid, w2_ref[...],
        dimension_numbers=(((1,), (1,)), ((), ())),
        preferred_element_type=jnp.float32) + b2_ref[...]

    content = jax.lax.dot_general(
        h_blk, wc_ref[...],
        dimension_numbers=(((1,), (1,)), ((), ())),
        preferred_element_type=jnp.float32) + bc_ref[...]

    mix = jax.nn.sigmoid(alpha_ref[0, 0])
    logits = mix * content + (1.0 - mix) * topo
    zmax = jnp.max(logits, axis=1, keepdims=True)
    ez = jnp.exp(logits - zmax)
    weights = ez / jnp.sum(ez, axis=1, keepdims=True)

    w_out_ref[...] = weights
    l_out_ref[...] = logits


def kernel(H, Wc, bc, W1, b1, W2, b2, alpha):
    b, s, d = H.shape
    n = b * s
    ng = Wc.shape[0]
    nh = W1.shape[0]
    k = min(8, n - 1)

    block_rows = 512
    while n % block_rows != 0:
        block_rows //= 2
    grid = n // block_rows

    hf = H.reshape(n, d)

    norm_rows = min(n, 1024)
    hn = pl.pallas_call(
        _normalize_kernel,
        grid=(n // norm_rows,),
        in_specs=[pl.BlockSpec((norm_rows, d), lambda i: (i, 0))],
        out_specs=pl.BlockSpec((norm_rows, d), lambda i: (i, 0)),
        out_shape=jax.ShapeDtypeStruct((n, d), jnp.float32),
    )(hf)

    body = functools.partial(_router_kernel, block_rows=block_rows, k=k)
    weights, logits = pl.pallas_call(
        body,
        grid=(grid,),
        in_specs=[
            pl.BlockSpec((block_rows, d), lambda i: (i, 0)),
            pl.BlockSpec((n, d), lambda i: (0, 0)),
            pl.BlockSpec((block_rows, d), lambda i: (i, 0)),
            pl.BlockSpec((ng, d), lambda i: (0, 0)),
            pl.BlockSpec((1, ng), lambda i: (0, 0)),
            pl.BlockSpec((nh, 3), lambda i: (0, 0)),
            pl.BlockSpec((1, nh), lambda i: (0, 0)),
            pl.BlockSpec((ng, nh), lambda i: (0, 0)),
            pl.BlockSpec((1, ng), lambda i: (0, 0)),
            pl.BlockSpec((1, 1), lambda i: (0, 0)),
        ],
        out_specs=[
            pl.BlockSpec((block_rows, ng), lambda i: (i, 0)),
            pl.BlockSpec((block_rows, ng), lambda i: (i, 0)),
        ],
        out_shape=[
            jax.ShapeDtypeStruct((n, ng), jnp.float32),
            jax.ShapeDtypeStruct((n, ng), jnp.float32),
        ],
        scratch_shapes=[
            pltpu.VMEM((block_rows, n), jnp.float32),
            pltpu.VMEM((block_rows, (k + 1) * 128), jnp.float32),
        ],
    )(hf, hn, hn, Wc, bc.reshape(1, ng), W1, b1.reshape(1, nh), W2,
      b2.reshape(1, ng), alpha.reshape(1, 1))

    return weights.reshape(b, s, ng), logits.reshape(b, s, ng)
